# baseline (speedup 1.0000x reference)
import jax
import jax.numpy as jnp
from jax import lax
from jax.experimental import pallas as pl
from jax.experimental.pallas import tpu as pltpu
from jax.experimental.pallas import tpu_sc as plsc

_B, _NT, _NS, _NOS, _FFT = 64, 4, 2, 14, 4096
_ROWS = _B * _NT * _NS * _NOS
_LEFT = 410
_DC = 2048
_W = _DC - _LEFT
_OUT_W = 2 * _W

_ALo = 408
_AW = 3280
_SHL = _LEFT - _ALo
_SHR = _DC + 1 - _ALo

_NWORKERS = 32
_ROWS_PER_W = _ROWS // _NWORKERS
_RBLK = 4
_NBLK = _ROWS_PER_W // _RBLK
_NBUF = 3
_NFULL = _W // 16


def _body(in_hbm, out_hbm, ibuf, obuf, sem_r, sem_w):
    wid = lax.axis_index("s") * 2 + lax.axis_index("c")
    base = wid * _ROWS_PER_W

    def read(i):
        b = i % _NBUF
        row = base + i * _RBLK
        return pltpu.make_async_copy(
            in_hbm.at[pl.ds(row, _RBLK), pl.ds(_ALo, _AW)],
            ibuf.at[b], sem_r.at[b])

    def write(i):
        b = i % _NBUF
        row = base + i * _RBLK
        return pltpu.make_async_copy(
            obuf.at[b], out_hbm.at[pl.ds(row, _RBLK), :], sem_w.at[b])

    def compute(i):
        b = i % _NBUF

        def jbody(j, carry):
            off = j * 16
            for r in range(_RBLK):
                obuf[b, r, pl.ds(off, 16)] = ibuf[b, r, pl.ds(_SHL + off, 16)]
                obuf[b, r, pl.ds(_W + off, 16)] = (
                    ibuf[b, r, pl.ds(_SHR + off, 16)])
            return carry

        lax.fori_loop(0, _NFULL, jbody, 0, unroll=2)
        for r in range(_RBLK):
            obuf[b, r, pl.ds(_W - 16, 16)] = (
                ibuf[b, r, pl.ds(_SHL + _W - 16, 16)])
            obuf[b, r, pl.ds(_OUT_W - 16, 16)] = (
                ibuf[b, r, pl.ds(_SHR + _W - 16, 16)])

    for i in range(min(_NBUF - 1, _NBLK)):
        read(i).start()
    for i in range(_NBLK):
        read(i).wait()
        compute(i)
        write(i).start()
        nxt = i + _NBUF - 1
        if nxt < _NBLK:
            if i >= 1:
                write(i - 1).wait()
            read(nxt).start()
    for i in range(max(0, _NBLK - _NBUF + 1), _NBLK):
        write(i).wait()


@jax.jit
def kernel(inputs, sc_ind):
    del sc_ind
    x = inputs.reshape(_ROWS, _FFT)
    run = pl.kernel(
        _body,
        out_type=jax.ShapeDtypeStruct((_ROWS, _OUT_W), jnp.float32),
        mesh=plsc.VectorSubcoreMesh(core_axis_name="c", subcore_axis_name="s"),
        compiler_params=pltpu.CompilerParams(use_tc_tiling_on_sc=False),
        scratch_types=[
            pltpu.VMEM((_NBUF, _RBLK, _AW), jnp.float32),
            pltpu.VMEM((_NBUF, _RBLK, _OUT_W), jnp.float32),
            pltpu.SemaphoreType.DMA((_NBUF,)),
            pltpu.SemaphoreType.DMA((_NBUF,)),
        ],
    )
    out = run(x)
    return out.reshape(_B, _NT, _NS, _NOS, _OUT_W)
